# Initial kernel scaffold; baseline (speedup 1.0000x reference)
import jax
import jax.numpy as jnp
from jax.experimental import pallas as pl


def kernel(x, edge_index, edge_weight, roi_num, batch, device, W1, b1, W11, b11, W2, b2, w4, b4, Wl1, bl1, Wl3, bl3, Wl11, bl11, Wl33, bl33, Wl4, bl4, Wl5, bl5, Wl6, bl6, Wl7, bl7):
    raise NotImplementedError("write your pallas kernel here")


# speedup vs baseline: 180.2247x; 180.2247x over previous
import functools

import jax
import jax.numpy as jnp
from jax import lax
from jax.experimental import pallas as pl
from jax.experimental.pallas import tpu as pltpu
from jax.experimental.pallas import tpu_sc as plsc

ROI = 116
LENN = 6670
BG = 128
SEG2 = 2 * LENN + ROI
HID = 64
HC2 = 128
NP = 128
N2P = 2 * NP
EP = 6672
AW = NP * NP
NTASK = 2 * BG
NC, NS = 2, 16
NW = NC * NS
TPW = NTASK // NW
NEG = -1e30


@functools.cache
def _sc_build_fn():
    mesh = plsc.VectorSubcoreMesh(core_axis_name="c", subcore_axis_name="s",
                                  num_cores=NC, num_subcores=NS)

    @functools.partial(
        pl.kernel,
        out_type=jax.ShapeDtypeStruct((NTASK, AW), jnp.float32),
        mesh=mesh,
        scratch_types=[
            pltpu.VMEM((EP,), jnp.int32),
            pltpu.VMEM((EP,), jnp.float32),
            pltpu.VMEM((AW,), jnp.float32),
        ],
        compiler_params=pltpu.CompilerParams(needs_layout_passes=False),
    )
    def _sc_build(idx_hbm, w_hbm, zero_hbm, out_hbm, idx_v, w_v, acc_v):
        wid = lax.axis_index("s") * NC + lax.axis_index("c")

        def task(r, carry):
            t = wid * TPW + r
            pltpu.sync_copy(idx_hbm.at[t], idx_v)
            pltpu.sync_copy(w_hbm.at[t], w_v)
            pltpu.sync_copy(zero_hbm, acc_v)

            def scat(i, c):
                sl = pl.ds(i * 16, 16)
                plsc.addupdate_scatter(acc_v, [idx_v[sl]], w_v[sl])
                return c

            lax.fori_loop(0, EP // 16, scat, 0)
            pltpu.sync_copy(acc_v, out_hbm.at[t])
            return carry

        lax.fori_loop(0, TPW, task, 0)

    return _sc_build


def _prop(A, h):
    deg = jnp.sum(A, axis=1, keepdims=True) + 1.0
    pos = deg > 0.0
    dinv = jnp.where(pos, lax.rsqrt(jnp.where(pos, deg, 1.0)), 0.0)
    hs = dinv * h
    t = jnp.dot(A, hs, preferred_element_type=jnp.float32)
    return dinv * (t + hs)


def _tc1_body(xp, ab, W1p, W11p, W2, w4c, b4s, b1r, b11r, b2r, Wl4T, bl4c,
              Wl5T, bl5c, z1_o, z2_o, xc_o, alls_o):
    f32 = jnp.float32
    x_fc = xp[0, 0]
    x_sc = xp[0, 1]
    A1 = ab[0, 0]
    A2 = ab[1, 0]

    h1 = jnp.dot(x_fc, W1p[...], preferred_element_type=f32)
    h2 = jnp.dot(x_sc, W11p[...], preferred_element_type=f32)
    rmask = lax.broadcasted_iota(jnp.int32, (NP, HID), 0) < ROI
    z1 = jnp.where(rmask, jax.nn.relu(_prop(A1, h1) + b1r[...]), 0.0)
    z2 = jnp.where(rmask, jax.nn.relu(_prop(A2, h2) + b11r[...]), 0.0)

    zcat = jnp.concatenate([z1, z2], axis=1)
    alls0 = jnp.dot(zcat, w4c[...], preferred_element_type=f32) + b4s[...]
    r1 = jax.nn.relu(jnp.dot(Wl4T[...], alls0, preferred_element_type=f32)
                     + bl4c[...])
    allsc = jax.nn.relu(jnp.dot(Wl5T[...], r1, preferred_element_type=f32)
                        + bl5c[...])

    eye = (lax.broadcasted_iota(jnp.int32, (NP, NP), 0)
           == lax.broadcasted_iota(jnp.int32, (NP, NP), 1))
    Dfu = jnp.where(eye, allsc, 0.0)
    zblk = jnp.zeros((NP, NP), f32)
    A3 = jnp.concatenate(
        [jnp.concatenate([A1, zblk], axis=1),
         jnp.concatenate([Dfu, A2], axis=1)], axis=0)

    zall = jnp.concatenate([z1, z2], axis=0)
    h3 = jnp.dot(zall, W2[...], preferred_element_type=f32)
    xc = jax.nn.relu(_prop(A3, h3) + b2r[...])
    ri = lax.broadcasted_iota(jnp.int32, (N2P, HID), 0)
    valid = (ri < ROI) | ((ri >= NP) & (ri < NP + ROI))
    xc = jnp.where(valid, xc, 0.0)

    z1_o[0] = z1
    z2_o[0] = z2
    xc_o[0] = xc
    alls_o[0] = jnp.sum(Dfu, axis=0, keepdims=True)


def _h12_body(z_ref, w_ref, b_ref, w3_ref, b3_ref, o_ref):
    f32 = jnp.float32
    H = jax.nn.relu(jnp.dot(z_ref[0], w_ref[0], preferred_element_type=f32)
                    + b_ref[0])
    L = jnp.dot(H, w3_ref[0], preferred_element_type=f32) + b3_ref[0]
    m = jnp.max(L, axis=1, keepdims=True)
    e = jnp.exp(L - m)
    o_ref[0] = e / jnp.sum(e, axis=1, keepdims=True)


def _h3_body(x_ref, w_ref, b6_ref, w7_ref, b7_ref, o_ref):
    f32 = jnp.float32
    k = pl.program_id(0)

    @pl.when(k == 0)
    def _():
        o_ref[...] = jnp.zeros_like(o_ref)

    o_ref[...] += jnp.dot(x_ref[...], w_ref[...], preferred_element_type=f32)

    @pl.when(k == pl.num_programs(0) - 1)
    def _():
        H = jax.nn.relu(o_ref[...] + b6_ref[...])
        L = jnp.dot(H, w7_ref[...], preferred_element_type=f32) + b7_ref[...]
        m = jnp.max(L, axis=1, keepdims=True)
        e = jnp.exp(L - m)
        o_ref[...] = e / jnp.sum(e, axis=1, keepdims=True)


def kernel(x, edge_index, edge_weight, roi_num, batch, device, W1, b1, W11,
           b11, W2, b2, w4, b4, Wl1, bl1, Wl3, bl3, Wl11, bl11, Wl33, bl33,
           Wl4, bl4, Wl5, bl5, Wl6, bl6, Wl7, bl7):
    del roi_num, batch, device
    f32 = jnp.float32

    ei = edge_index.astype(jnp.int32).reshape(2, BG, SEG2)
    ew = edge_weight.reshape(BG, SEG2)
    loc = ei - (jnp.arange(BG, dtype=jnp.int32) * (2 * ROI))[None, :, None]
    flat_fc = loc[1, :, :LENN] * NP + loc[0, :, :LENN]
    flat_sc = ((loc[1, :, LENN:2 * LENN] - ROI) * NP
               + (loc[0, :, LENN:2 * LENN] - ROI))
    idx2 = jnp.pad(jnp.concatenate([flat_fc, flat_sc], axis=0),
                   ((0, 0), (0, EP - LENN)))
    w2 = jnp.pad(jnp.concatenate([ew[:, :LENN], ew[:, LENN:2 * LENN]], axis=0),
                 ((0, 0), (0, EP - LENN)))
    zero_row = jnp.zeros((AW,), f32)

    a_flat = _sc_build_fn()(idx2, w2, zero_row)
    ab = a_flat.reshape(2, BG, NP, NP)

    xp = jnp.pad(x.reshape(BG, 2, ROI, ROI - 1),
                 ((0, 0), (0, 0), (0, NP - ROI), (0, NP - (ROI - 1))))
    W1p = jnp.pad(W1, ((0, NP - (ROI - 1)), (0, 0)))
    W11p = jnp.pad(W11, ((0, NP - (ROI - 1)), (0, 0)))
    w4c = w4.reshape(2 * HID, 1)
    b4s = b4.reshape(1, 1)
    b1r = b1.reshape(1, HID)
    b11r = b11.reshape(1, HID)
    b2r = b2.reshape(1, HID)
    Wl4T = jnp.pad(Wl4.T, ((0, 0), (0, NP - ROI)))
    bl4c = bl4.reshape(HID, 1)
    Wl5T = jnp.pad(Wl5.T, ((0, NP - ROI), (0, 0)))
    bl5c = jnp.pad(bl5, (0, NP - ROI)).reshape(NP, 1)

    cst = lambda *shape: pl.BlockSpec(shape, lambda g: (0,) * len(shape))
    z1o, z2o, xco, allso = pl.pallas_call(
        _tc1_body,
        grid=(BG,),
        in_specs=[
            pl.BlockSpec((1, 2, NP, NP), lambda g: (g, 0, 0, 0)),
            pl.BlockSpec((2, 1, NP, NP), lambda g: (0, g, 0, 0)),
            cst(NP, HID), cst(NP, HID), cst(HID, HID), cst(2 * HID, 1),
            cst(1, 1), cst(1, HID), cst(1, HID), cst(1, HID),
            cst(HID, NP), cst(HID, 1), cst(NP, HID), cst(NP, 1),
        ],
        out_specs=[
            pl.BlockSpec((1, NP, HID), lambda g: (g, 0, 0)),
            pl.BlockSpec((1, NP, HID), lambda g: (g, 0, 0)),
            pl.BlockSpec((1, N2P, HID), lambda g: (g, 0, 0)),
            pl.BlockSpec((1, 1, NP), lambda g: (g, 0, 0)),
        ],
        out_shape=[
            jax.ShapeDtypeStruct((BG, NP, HID), f32),
            jax.ShapeDtypeStruct((BG, NP, HID), f32),
            jax.ShapeDtypeStruct((BG, N2P, HID), f32),
            jax.ShapeDtypeStruct((BG, 1, NP), f32),
        ],
    )(xp, ab, W1p, W11p, W2, w4c, b4s, b1r, b11r, b2r, Wl4T, bl4c, Wl5T, bl5c)

    ZF = NP * HID
    Z = jnp.stack([z1o.reshape(BG, ZF), z2o.reshape(BG, ZF)])
    padw = lambda W: jnp.pad(W.reshape(ROI, HID, HC2),
                             ((0, NP - ROI), (0, 0), (0, 0))).reshape(ZF, HC2)
    ZW = jnp.stack([padw(Wl1), padw(Wl11)])
    b1s = jnp.stack([bl1, bl11]).reshape(2, 1, HC2)
    W3s = jnp.stack([jnp.pad(Wl3, ((0, 0), (0, HC2 - 2))),
                     jnp.pad(Wl33, ((0, 0), (0, HC2 - 2)))])
    b3s = jnp.stack([jnp.pad(bl3, (0, HC2 - 2), constant_values=NEG),
                     jnp.pad(bl33, (0, HC2 - 2), constant_values=NEG)]
                    ).reshape(2, 1, HC2)
    h12 = pl.pallas_call(
        _h12_body,
        grid=(2,),
        in_specs=[
            pl.BlockSpec((1, BG, ZF), lambda i: (i, 0, 0)),
            pl.BlockSpec((1, ZF, HC2), lambda i: (i, 0, 0)),
            pl.BlockSpec((1, 1, HC2), lambda i: (i, 0, 0)),
            pl.BlockSpec((1, HC2, HC2), lambda i: (i, 0, 0)),
            pl.BlockSpec((1, 1, HC2), lambda i: (i, 0, 0)),
        ],
        out_specs=pl.BlockSpec((1, BG, HC2), lambda i: (i, 0, 0)),
        out_shape=jax.ShapeDtypeStruct((2, BG, HC2), f32),
    )(Z, ZW, b1s, W3s, b3s)

    XF = N2P * HID
    XCf = xco.reshape(BG, XF)
    Wl6r = Wl6.reshape(2 * ROI, HID, HC2)
    Wl6p = (jnp.zeros((N2P, HID, HC2), f32)
            .at[:ROI].set(Wl6r[:ROI])
            .at[NP:NP + ROI].set(Wl6r[ROI:])).reshape(XF, HC2)
    bl6r = bl6.reshape(1, HC2)
    Wl7p = jnp.pad(Wl7, ((0, 0), (0, HC2 - 2)))
    bl7p = jnp.pad(bl7, (0, HC2 - 2), constant_values=NEG).reshape(1, HC2)
    KCH = 4
    KB = XF // KCH
    xfp = pl.pallas_call(
        _h3_body,
        grid=(KCH,),
        in_specs=[
            pl.BlockSpec((BG, KB), lambda k: (0, k)),
            pl.BlockSpec((KB, HC2), lambda k: (k, 0)),
            pl.BlockSpec((1, HC2), lambda k: (0, 0)),
            pl.BlockSpec((HC2, HC2), lambda k: (0, 0)),
            pl.BlockSpec((1, HC2), lambda k: (0, 0)),
        ],
        out_specs=pl.BlockSpec((BG, HC2), lambda k: (0, 0)),
        out_shape=jax.ShapeDtypeStruct((BG, HC2), f32),
    )(XCf, Wl6p, bl6r, Wl7p, bl7p)

    xf = xfp[:, :2]
    x1 = h12[0, :, :2]
    x2 = h12[1, :, :2]
    alls = allso.reshape(BG, NP)[:, :ROI]
    return (xf, x1, x2, alls)


# speedup vs baseline: 181.8521x; 1.0090x over previous
import functools

import jax
import jax.numpy as jnp
from jax import lax
from jax.experimental import pallas as pl
from jax.experimental.pallas import tpu as pltpu
from jax.experimental.pallas import tpu_sc as plsc

ROI = 116
LENN = 6670
BG = 128
SEG2 = 2 * LENN + ROI
NE = BG * SEG2
HID = 64
HC2 = 128
NP = 128
N2P = 2 * NP
ECH = 6688
AW = NP * NP
NTASK = 2 * BG
NC, NS = 2, 16
NW = NC * NS
TPW = NTASK // NW
NEG = -1e30


@functools.cache
def _sc_build_fn():
    mesh = plsc.VectorSubcoreMesh(core_axis_name="c", subcore_axis_name="s",
                                  num_cores=NC, num_subcores=NS)

    @functools.partial(
        pl.kernel,
        out_type=jax.ShapeDtypeStruct((NTASK, AW), jnp.float32),
        mesh=mesh,
        scratch_types=[
            pltpu.VMEM((ECH,), jnp.int32),
            pltpu.VMEM((ECH,), jnp.int32),
            pltpu.VMEM((ECH,), jnp.float32),
            pltpu.VMEM((AW,), jnp.float32),
        ],
        compiler_params=pltpu.CompilerParams(needs_layout_passes=False),
    )
    def _sc_build(ei_hbm, ew_hbm, zero_hbm, out_hbm, src_v, dst_v, w_v, acc_v):
        wid = lax.axis_index("s") * NC + lax.axis_index("c")

        def task(r, carry):
            t = wid * TPW + r
            p = t // BG
            g = t - p * BG
            base = g * SEG2 + p * LENN
            off = 6 * p
            abase = pl.multiple_of(base - off, 8)
            pltpu.sync_copy(ei_hbm.at[pl.ds(abase, ECH)], src_v)
            pltpu.sync_copy(ei_hbm.at[pl.ds(NE + abase, ECH)], dst_v)
            pltpu.sync_copy(ew_hbm.at[pl.ds(abase, ECH)], w_v)
            pltpu.sync_copy(zero_hbm, acc_v)
            cc = 129 * (g * (2 * ROI) + ROI * p)
            lo = off
            hi = off + LENN

            def scat(i, c):
                sl = pl.ds(i * 16, 16)
                pos = i * 16 + lax.iota(jnp.int32, 16)
                valid = (pos >= lo) & (pos < hi)
                idx = dst_v[sl] * NP + src_v[sl] - cc
                idx = jnp.where(valid, idx, 0)
                w = jnp.where(valid, w_v[sl], 0.0)
                plsc.addupdate_scatter(acc_v, [idx], w)
                return c

            lax.fori_loop(0, ECH // 16, scat, 0)
            pltpu.sync_copy(acc_v, out_hbm.at[t])
            return carry

        lax.fori_loop(0, TPW, task, 0)

    return _sc_build


def _prop(A, h):
    deg = jnp.sum(A, axis=1, keepdims=True) + 1.0
    pos = deg > 0.0
    dinv = jnp.where(pos, lax.rsqrt(jnp.where(pos, deg, 1.0)), 0.0)
    hs = dinv * h
    t = jnp.dot(A, hs, preferred_element_type=jnp.float32)
    return dinv * (t + hs)


def _tc1_body(xr, ab, W1, W11, W2, w4c, b4s, b1r, b11r, b2r, Wl4T, bl4c,
              Wl5T, bl5c, z_o, xc_o, alls_o):
    f32 = jnp.float32
    zpad = jnp.zeros((NP - ROI, HID), f32)

    def branch(xblk, A, W, br):
        h = jnp.concatenate(
            [jnp.dot(xblk, W, preferred_element_type=f32), zpad], axis=0)
        z = jax.nn.relu(_prop(A, h) + br)
        return jnp.concatenate([z[:ROI], zpad], axis=0)

    z1 = branch(xr[0, 0], ab[0, 0], W1[...], b1r[...])
    z2 = branch(xr[0, 1], ab[1, 0], W11[...], b11r[...])

    zcat = jnp.concatenate([z1, z2], axis=1)
    alls0 = jnp.dot(zcat, w4c[...], preferred_element_type=f32) + b4s[...]
    r1 = jax.nn.relu(
        jnp.dot(Wl4T[...], alls0[:ROI], preferred_element_type=f32)
        + bl4c[...])
    r2 = jax.nn.relu(jnp.dot(Wl5T[...], r1, preferred_element_type=f32)
                     + bl5c[...])
    allsc = jnp.concatenate([r2, jnp.zeros((NP - ROI, 1), f32)], axis=0)

    eye = (lax.broadcasted_iota(jnp.int32, (NP, NP), 0)
           == lax.broadcasted_iota(jnp.int32, (NP, NP), 1))
    Dfu = jnp.where(eye, allsc, 0.0)
    zblk = jnp.zeros((NP, NP), f32)
    A3 = jnp.concatenate(
        [jnp.concatenate([ab[0, 0], zblk], axis=1),
         jnp.concatenate([Dfu, ab[1, 0]], axis=1)], axis=0)

    zall = jnp.concatenate([z1, z2], axis=0)
    h3 = jnp.dot(zall, W2[...], preferred_element_type=f32)
    xc = jax.nn.relu(_prop(A3, h3) + b2r[...])

    z_o[0, 0] = z1[:ROI]
    z_o[1, 0] = z2[:ROI]
    xc_o[0] = jnp.concatenate([xc[:ROI], xc[NP:NP + ROI]], axis=0)
    alls_o[0] = jnp.sum(Dfu, axis=0, keepdims=True)


def _head_body(z_ref, w_ref, b_ref, w3_ref, b3_ref, o_ref):
    f32 = jnp.float32
    H = jax.nn.relu(jnp.dot(z_ref[...], w_ref[...], preferred_element_type=f32)
                    + b_ref[...])
    L = jnp.dot(H, w3_ref[...], preferred_element_type=f32) + b3_ref[...]
    m = jnp.max(L, axis=1, keepdims=True)
    e = jnp.exp(L - m)
    o_ref[...] = e / jnp.sum(e, axis=1, keepdims=True)


def _h3_body(x_ref, w_ref, b6_ref, w7_ref, b7_ref, o_ref):
    f32 = jnp.float32
    k = pl.program_id(0)

    @pl.when(k == 0)
    def _():
        o_ref[...] = jnp.zeros_like(o_ref)

    o_ref[...] += jnp.dot(x_ref[...], w_ref[...], preferred_element_type=f32)

    @pl.when(k == pl.num_programs(0) - 1)
    def _():
        H = jax.nn.relu(o_ref[...] + b6_ref[...])
        L = jnp.dot(H, w7_ref[...], preferred_element_type=f32) + b7_ref[...]
        m = jnp.max(L, axis=1, keepdims=True)
        e = jnp.exp(L - m)
        o_ref[...] = e / jnp.sum(e, axis=1, keepdims=True)


def _softmax_head(Z, W, b, W3, b3):
    f32 = jnp.float32
    K = Z.shape[1]
    W3p = jnp.pad(W3, ((0, 0), (0, HC2 - W3.shape[1])))
    b3p = jnp.pad(b3, (0, HC2 - b3.shape[0]), constant_values=NEG
                  ).reshape(1, HC2)
    return pl.pallas_call(
        _head_body,
        out_shape=jax.ShapeDtypeStruct((BG, HC2), f32),
    )(Z, W, b.reshape(1, HC2), W3p, b3p)


def kernel(x, edge_index, edge_weight, roi_num, batch, device, W1, b1, W11,
           b11, W2, b2, w4, b4, Wl1, bl1, Wl3, bl3, Wl11, bl11, Wl33, bl33,
           Wl4, bl4, Wl5, bl5, Wl6, bl6, Wl7, bl7):
    del roi_num, batch, device
    f32 = jnp.float32

    ei = edge_index.astype(jnp.int32).reshape(2 * NE)
    zero_row = jnp.zeros((AW,), f32)
    a_flat = _sc_build_fn()(ei, edge_weight, zero_row)
    ab = a_flat.reshape(2, BG, NP, NP)

    xr = x.reshape(BG, 2, ROI, ROI - 1)
    cst = lambda *shape: pl.BlockSpec(shape, lambda g: (0,) * len(shape))
    zo, xco, allso = pl.pallas_call(
        _tc1_body,
        grid=(BG,),
        in_specs=[
            pl.BlockSpec((1, 2, ROI, ROI - 1), lambda g: (g, 0, 0, 0)),
            pl.BlockSpec((2, 1, NP, NP), lambda g: (0, g, 0, 0)),
            cst(ROI - 1, HID), cst(ROI - 1, HID), cst(HID, HID),
            cst(2 * HID, 1), cst(1, 1), cst(1, HID), cst(1, HID),
            cst(1, HID), cst(HID, ROI), cst(HID, 1), cst(ROI, HID),
            cst(ROI, 1),
        ],
        out_specs=[
            pl.BlockSpec((2, 1, ROI, HID), lambda g: (0, g, 0, 0)),
            pl.BlockSpec((1, 2 * ROI, HID), lambda g: (g, 0, 0)),
            pl.BlockSpec((1, 1, NP), lambda g: (g, 0, 0)),
        ],
        out_shape=[
            jax.ShapeDtypeStruct((2, BG, ROI, HID), f32),
            jax.ShapeDtypeStruct((BG, 2 * ROI, HID), f32),
            jax.ShapeDtypeStruct((BG, 1, NP), f32),
        ],
    )(xr, ab, W1, W11, W2, w4.reshape(2 * HID, 1), b4.reshape(1, 1),
      b1.reshape(1, HID), b11.reshape(1, HID), b2.reshape(1, HID),
      Wl4.T, bl4.reshape(HID, 1), Wl5.T, bl5.reshape(ROI, 1))

    x1 = _softmax_head(zo[0].reshape(BG, ROI * HID), Wl1, bl1, Wl3, bl3)
    x2 = _softmax_head(zo[1].reshape(BG, ROI * HID), Wl11, bl11, Wl33, bl33)

    XF = 2 * ROI * HID
    KCH = 4
    KB = XF // KCH
    Wl7p = jnp.pad(Wl7, ((0, 0), (0, HC2 - 2)))
    bl7p = jnp.pad(bl7, (0, HC2 - 2), constant_values=NEG).reshape(1, HC2)
    xfp = pl.pallas_call(
        _h3_body,
        grid=(KCH,),
        in_specs=[
            pl.BlockSpec((BG, KB), lambda k: (0, k)),
            pl.BlockSpec((KB, HC2), lambda k: (k, 0)),
            pl.BlockSpec((1, HC2), lambda k: (0, 0)),
            pl.BlockSpec((HC2, HC2), lambda k: (0, 0)),
            pl.BlockSpec((1, HC2), lambda k: (0, 0)),
        ],
        out_specs=pl.BlockSpec((BG, HC2), lambda k: (0, 0)),
        out_shape=jax.ShapeDtypeStruct((BG, HC2), f32),
    )(xco.reshape(BG, XF), Wl6, bl6.reshape(1, HC2), Wl7p, bl7p)

    xf = xfp[:, :2]
    x1 = x1[:, :2]
    x2 = x2[:, :2]
    alls = allso.reshape(BG, NP)[:, :ROI]
    return (xf, x1, x2, alls)


# speedup vs baseline: 229.3349x; 1.2611x over previous
import functools

import jax
import jax.numpy as jnp
from jax import lax
from jax.experimental import pallas as pl
from jax.experimental.pallas import tpu as pltpu
from jax.experimental.pallas import tpu_sc as plsc

ROI = 116
LENN = 6670
BG = 128
SEG2 = 2 * LENN + ROI
NE = BG * SEG2
HID = 64
HC2 = 128
NP = 128
N2P = 2 * NP
ECH = 6688
AW = NP * NP
NTASK = 2 * BG
NC, NS = 2, 16
NW = NC * NS
TPW = NTASK // NW
GB = 4
NEG = -1e30


@functools.cache
def _sc_build_fn():
    mesh = plsc.VectorSubcoreMesh(core_axis_name="c", subcore_axis_name="s",
                                  num_cores=NC, num_subcores=NS)

    @functools.partial(
        pl.kernel,
        out_type=jax.ShapeDtypeStruct((NTASK, AW), jnp.float32),
        mesh=mesh,
        scratch_types=[
            pltpu.VMEM((2, ECH), jnp.int32),
            pltpu.VMEM((ECH,), jnp.float32),
            pltpu.VMEM((AW,), jnp.float32),
        ],
        compiler_params=pltpu.CompilerParams(needs_layout_passes=False,
                                             use_tc_tiling_on_sc=False),
    )
    def _sc_build(ei_hbm, ew_hbm, zero_hbm, out_hbm, sd_v, w_v, acc_v):
        wid = lax.axis_index("s") * NC + lax.axis_index("c")

        def task(r, carry):
            t = wid * TPW + r
            p = t // BG
            g = t - p * BG
            off = 6 * p
            abase = pl.multiple_of(g * SEG2 + p * LENN - off, 8)
            pltpu.sync_copy(ei_hbm.at[:, pl.ds(abase, ECH)], sd_v)
            pltpu.sync_copy(ew_hbm.at[pl.ds(abase, ECH)], w_v)
            pltpu.sync_copy(zero_hbm, acc_v)
            cc = 129 * (g * (2 * ROI) + ROI * p)
            lo = off
            hi = off + LENN

            @plsc.parallel_loop(0, ECH // 16, 1, unroll=8)
            def scat(i):
                sl = pl.ds(i * 16, 16)
                pos = i * 16 + lax.iota(jnp.int32, 16)
                valid = (pos >= lo) & (pos < hi)
                idx = sd_v[1, sl] * NP + sd_v[0, sl] - cc
                idx = jnp.where(valid, idx, 0)
                w = jnp.where(valid, w_v[sl], 0.0)
                plsc.addupdate_scatter(acc_v, [idx], w)

            pltpu.sync_copy(acc_v, out_hbm.at[t])
            return carry

        lax.fori_loop(0, TPW, task, 0)

    return _sc_build


def _prop(A, h):
    deg = jnp.sum(A, axis=1, keepdims=True) + 1.0
    pos = deg > 0.0
    dinv = jnp.where(pos, lax.rsqrt(jnp.where(pos, deg, 1.0)), 0.0)
    hs = dinv * h
    t = jnp.dot(A, hs, preferred_element_type=jnp.float32)
    return dinv * (t + hs)


def _tc1_body(xr, ab, W1, W11, W2, w4c, b4s, b1r, b11r, b2r, Wl4T, bl4c,
              Wl5T, bl5c, z_o, xc_o, alls_o):
    f32 = jnp.float32
    zpad = jnp.zeros((NP - ROI, HID), f32)
    eye = (lax.broadcasted_iota(jnp.int32, (NP, NP), 0)
           == lax.broadcasted_iota(jnp.int32, (NP, NP), 1))
    zblk = jnp.zeros((NP, NP), f32)
    rmask = lax.broadcasted_iota(jnp.int32, (NP, HID), 0) < ROI

    def _evenodd(n):
        r = lax.broadcasted_iota(jnp.int32, (n // 2, n), 0)
        c = lax.broadcasted_iota(jnp.int32, (n // 2, n), 1)
        return (jnp.where(c == 2 * r, 1.0, 0.0).astype(f32),
                jnp.where(c == 2 * r + 1, 1.0, 0.0).astype(f32))

    Se, So = _evenodd(NP)
    S3e, S3o = _evenodd(N2P)

    def _pair(S2, x2):
        return jnp.concatenate(
            [jnp.dot(S2[0], x2, preferred_element_type=f32),
             jnp.dot(S2[1], x2, preferred_element_type=f32)], axis=1)

    for i in range(GB):
        def branch(xblk, A, W, br):
            h = jnp.concatenate(
                [jnp.dot(xblk, W, preferred_element_type=f32), zpad], axis=0)
            z = jax.nn.relu(_prop(A, h) + br)
            return jnp.where(rmask, z, 0.0)

        A1 = ab[0, i]
        A2 = ab[1, i]
        z1 = branch(xr[i, 0], A1, W1[...], b1r[...])
        z2 = branch(xr[i, 1], A2, W11[...], b11r[...])

        zcat = jnp.concatenate([z1, z2], axis=1)
        alls0 = jnp.dot(zcat, w4c[...], preferred_element_type=f32) + b4s[...]
        r1 = jax.nn.relu(
            jnp.dot(Wl4T[...], alls0[:ROI], preferred_element_type=f32)
            + bl4c[...])
        r2 = jax.nn.relu(jnp.dot(Wl5T[...], r1, preferred_element_type=f32)
                         + bl5c[...])
        allsc = jnp.concatenate([r2, jnp.zeros((NP - ROI, 1), f32)], axis=0)

        Dfu = jnp.where(eye, allsc, 0.0)
        A3 = jnp.concatenate(
            [jnp.concatenate([A1, zblk], axis=1),
             jnp.concatenate([Dfu, A2], axis=1)], axis=0)

        zall = jnp.concatenate([z1, z2], axis=0)
        h3 = jnp.dot(zall, W2[...], preferred_element_type=f32)
        xc = jax.nn.relu(_prop(A3, h3) + b2r[...])
        ri = lax.broadcasted_iota(jnp.int32, (N2P, HID), 0)
        valid = (ri < ROI) | ((ri >= NP) & (ri < NP + ROI))
        xc = jnp.where(valid, xc, 0.0)

        z_o[0, i] = _pair((Se, So), z1)
        z_o[1, i] = _pair((Se, So), z2)
        xc_o[i] = _pair((S3e, S3o), xc)
        alls_o[i] = jnp.sum(Dfu, axis=0, keepdims=True)


def _head_body(z_ref, w_ref, b_ref, w3_ref, b3_ref, o_ref):
    f32 = jnp.float32
    H = jax.nn.relu(jnp.dot(z_ref[...], w_ref[...], preferred_element_type=f32)
                    + b_ref[...])
    L = jnp.dot(H, w3_ref[...], preferred_element_type=f32) + b3_ref[...]
    m = jnp.max(L, axis=1, keepdims=True)
    e = jnp.exp(L - m)
    o_ref[...] = e / jnp.sum(e, axis=1, keepdims=True)


def _h3_body(x_ref, w_ref, b6_ref, w7_ref, b7_ref, o_ref):
    f32 = jnp.float32
    k = pl.program_id(0)

    @pl.when(k == 0)
    def _():
        o_ref[...] = jnp.zeros_like(o_ref)

    o_ref[...] += jnp.dot(x_ref[...], w_ref[...], preferred_element_type=f32)

    @pl.when(k == pl.num_programs(0) - 1)
    def _():
        H = jax.nn.relu(o_ref[...] + b6_ref[...])
        L = jnp.dot(H, w7_ref[...], preferred_element_type=f32) + b7_ref[...]
        m = jnp.max(L, axis=1, keepdims=True)
        e = jnp.exp(L - m)
        o_ref[...] = e / jnp.sum(e, axis=1, keepdims=True)


def _softmax_head(Z, W, b, W3, b3):
    W3p = jnp.pad(W3, ((0, 0), (0, HC2 - W3.shape[1])))
    b3p = jnp.pad(b3, (0, HC2 - b3.shape[0]), constant_values=NEG
                  ).reshape(1, HC2)
    return pl.pallas_call(
        _head_body,
        out_shape=jax.ShapeDtypeStruct((BG, HC2), jnp.float32),
    )(Z, W, b.reshape(1, HC2), W3p, b3p)


def kernel(x, edge_index, edge_weight, roi_num, batch, device, W1, b1, W11,
           b11, W2, b2, w4, b4, Wl1, bl1, Wl3, bl3, Wl11, bl11, Wl33, bl33,
           Wl4, bl4, Wl5, bl5, Wl6, bl6, Wl7, bl7):
    del roi_num, batch, device
    f32 = jnp.float32

    ei = edge_index.astype(jnp.int32)
    zero_row = jnp.zeros((AW,), f32)
    a_flat = _sc_build_fn()(ei, edge_weight, zero_row)
    ab = a_flat.reshape(2, BG, NP, NP)

    xr = x.reshape(BG, 2, ROI, ROI - 1)
    cst = lambda *shape: pl.BlockSpec(shape, lambda g: (0,) * len(shape))
    zo, xco, allso = pl.pallas_call(
        _tc1_body,
        grid=(BG // GB,),
        in_specs=[
            pl.BlockSpec((GB, 2, ROI, ROI - 1), lambda g: (g, 0, 0, 0)),
            pl.BlockSpec((2, GB, NP, NP), lambda g: (0, g, 0, 0)),
            cst(ROI - 1, HID), cst(ROI - 1, HID), cst(HID, HID),
            cst(2 * HID, 1), cst(1, 1), cst(1, HID), cst(1, HID),
            cst(1, HID), cst(HID, ROI), cst(HID, 1), cst(ROI, HID),
            cst(ROI, 1),
        ],
        out_specs=[
            pl.BlockSpec((2, GB, HID, NP), lambda g: (0, g, 0, 0)),
            pl.BlockSpec((GB, NP, NP), lambda g: (g, 0, 0)),
            pl.BlockSpec((GB, 1, NP), lambda g: (g, 0, 0)),
        ],
        out_shape=[
            jax.ShapeDtypeStruct((2, BG, HID, NP), f32),
            jax.ShapeDtypeStruct((BG, NP, NP), f32),
            jax.ShapeDtypeStruct((BG, 1, NP), f32),
        ],
    )(xr, ab, W1, W11, W2, w4.reshape(2 * HID, 1), b4.reshape(1, 1),
      b1.reshape(1, HID), b11.reshape(1, HID), b2.reshape(1, HID),
      Wl4.T, bl4.reshape(HID, 1), Wl5.T, bl5.reshape(ROI, 1))

    ZF = NP * HID
    padw = lambda W: jnp.pad(W.reshape(ROI, HID, HC2),
                             ((0, NP - ROI), (0, 0), (0, 0))).reshape(ZF, HC2)
    x1 = _softmax_head(zo[0].reshape(BG, ZF), padw(Wl1), bl1, Wl3, bl3)
    x2 = _softmax_head(zo[1].reshape(BG, ZF), padw(Wl11), bl11, Wl33, bl33)

    XF = N2P * HID
    Wl6r = Wl6.reshape(2 * ROI, HID, HC2)
    Wl6p = (jnp.zeros((N2P, HID, HC2), f32)
            .at[:ROI].set(Wl6r[:ROI])
            .at[NP:NP + ROI].set(Wl6r[ROI:])).reshape(XF, HC2)
    KCH = 4
    KB = XF // KCH
    Wl7p = jnp.pad(Wl7, ((0, 0), (0, HC2 - 2)))
    bl7p = jnp.pad(bl7, (0, HC2 - 2), constant_values=NEG).reshape(1, HC2)
    xfp = pl.pallas_call(
        _h3_body,
        grid=(KCH,),
        in_specs=[
            pl.BlockSpec((BG, KB), lambda k: (0, k)),
            pl.BlockSpec((KB, HC2), lambda k: (k, 0)),
            pl.BlockSpec((1, HC2), lambda k: (0, 0)),
            pl.BlockSpec((HC2, HC2), lambda k: (0, 0)),
            pl.BlockSpec((1, HC2), lambda k: (0, 0)),
        ],
        out_specs=pl.BlockSpec((BG, HC2), lambda k: (0, 0)),
        out_shape=jax.ShapeDtypeStruct((BG, HC2), f32),
    )(xco.reshape(BG, XF), Wl6p, bl6.reshape(1, HC2), Wl7p, bl7p)

    xf = xfp[:, :2]
    x1 = x1[:, :2]
    x2 = x2[:, :2]
    alls = allso.reshape(BG, NP)[:, :ROI]
    return (xf, x1, x2, alls)


# speedup vs baseline: 256.1899x; 1.1171x over previous
import functools

import jax
import jax.numpy as jnp
from jax import lax
from jax.experimental import pallas as pl
from jax.experimental.pallas import tpu as pltpu
from jax.experimental.pallas import tpu_sc as plsc

ROI = 116
LENN = 6670
BG = 128
SEG2 = 2 * LENN + ROI
NE = BG * SEG2
HID = 64
HC2 = 128
NP = 128
N2P = 2 * NP
ECH = 6688
AW = NP * NP
NTASK = 2 * BG
NC, NS = 2, 16
NW = NC * NS
TPW = NTASK // NW
GB = 4
NEG = -1e30


@functools.cache
def _sc_build_fn():
    mesh = plsc.VectorSubcoreMesh(core_axis_name="c", subcore_axis_name="s",
                                  num_cores=NC, num_subcores=NS)

    @functools.partial(
        pl.kernel,
        out_type=jax.ShapeDtypeStruct((NTASK, AW), jnp.float32),
        mesh=mesh,
        scratch_types=[
            pltpu.VMEM((2, ECH), jnp.int32),
            pltpu.VMEM((ECH,), jnp.float32),
            pltpu.VMEM((AW,), jnp.float32),
        ],
        compiler_params=pltpu.CompilerParams(needs_layout_passes=False,
                                             use_tc_tiling_on_sc=False),
    )
    def _sc_build(ei_hbm, ew_hbm, zero_hbm, out_hbm, sd_v, w_v, acc_v):
        wid = lax.axis_index("s") * NC + lax.axis_index("c")

        def task(r, carry):
            t = wid * TPW + r
            p = t // BG
            g = t - p * BG
            off = 6 * p
            abase = pl.multiple_of(g * SEG2 + p * LENN - off, 8)
            pltpu.sync_copy(ei_hbm.at[:, pl.ds(abase, ECH)], sd_v)
            pltpu.sync_copy(ew_hbm.at[pl.ds(abase, ECH)], w_v)
            pltpu.sync_copy(zero_hbm, acc_v)
            cc = 129 * (g * (2 * ROI) + ROI * p) - 516 * p
            lo = off
            hi = off + LENN

            @plsc.parallel_loop(0, ECH // 16, 1, unroll=8)
            def scat(i):
                sl = pl.ds(i * 16, 16)
                pos = i * 16 + lax.iota(jnp.int32, 16)
                valid = (pos >= lo) & (pos < hi)
                idx = sd_v[1, sl] * NP + sd_v[0, sl] - cc
                idx = jnp.where(valid, idx, 0)
                w = jnp.where(valid, w_v[sl], 0.0)
                plsc.addupdate_scatter(acc_v, [idx], w)

            pltpu.sync_copy(acc_v, out_hbm.at[t])
            return carry

        lax.fori_loop(0, TPW, task, 0)

    return _sc_build


def _prop(A, h):
    deg = jnp.sum(A, axis=1, keepdims=True) + 1.0
    dinv = lax.rsqrt(deg)
    hs = dinv * h
    t = jnp.dot(A, hs, preferred_element_type=jnp.float32)
    return dinv * (t + hs)


def _tc1_body(x2d, ab, W1, W11, W2, w4c, b4s, b1r, b11r, b2r, Wl4T, bl4c,
              Wl5T, bl5c, z_o, xc_o, alls_o):
    f32 = jnp.float32
    one = jnp.float32(1.0)
    nil = jnp.float32(0.0)
    rr = lax.broadcasted_iota(jnp.int32, (NP, NP), 0)
    cc = lax.broadcasted_iota(jnp.int32, (NP, NP), 1)
    eye = rr == cc
    zblk = jnp.zeros((NP, NP), f32)
    hpad = jnp.zeros((8, HID), f32)
    rih = lax.broadcasted_iota(jnp.int32, (NP, HID), 0)
    rmask1 = rih < ROI
    rmask2 = (rih >= 4) & (rih < 4 + ROI)
    r6 = lax.broadcasted_iota(jnp.int32, (HID, NP), 0)
    c6 = lax.broadcasted_iota(jnp.int32, (HID, NP), 1)
    Se = jnp.where(c6 == 2 * r6, one, nil)
    So = jnp.where(c6 == 2 * r6 + 1, one, nil)
    S2e = jnp.where(c6 == 2 * r6 + 4, one, nil)
    S2o = jnp.where(c6 == 2 * r6 + 5, one, nil)
    r3 = lax.broadcasted_iota(jnp.int32, (NP, N2P), 0)
    c3 = lax.broadcasted_iota(jnp.int32, (NP, N2P), 1)
    sel3 = 2 * r3 + jnp.where(r3 >= ROI // 2, 16, 0)
    C3e = jnp.where(c3 == sel3, one, nil)
    C3o = jnp.where(c3 == sel3 + 1, one, nil)
    Sh4u = jnp.where(cc == rr + 4, one, nil)

    for i in range(GB):
        A1 = ab[0, i]
        A2 = ab[1, i]
        xf_blk = x2d[pl.ds(232 * i, 120), :]
        xs_blk = x2d[pl.ds(232 * i + 112, 120), :]
        h1 = jnp.concatenate(
            [jnp.dot(xf_blk, W1[...], preferred_element_type=f32), hpad],
            axis=0)
        h2 = jnp.concatenate(
            [jnp.dot(xs_blk, W11[...], preferred_element_type=f32), hpad],
            axis=0)
        z1 = jnp.where(rmask1, jax.nn.relu(_prop(A1, h1) + b1r[...]), nil)
        z2 = jnp.where(rmask2, jax.nn.relu(_prop(A2, h2) + b11r[...]), nil)
        z2c = jnp.dot(Sh4u, z2, preferred_element_type=f32)

        zcat = jnp.concatenate([z1, z2c], axis=1)
        alls0 = jnp.dot(zcat, w4c[...], preferred_element_type=f32) + b4s[...]
        r1 = jax.nn.relu(
            jnp.dot(Wl4T[...], alls0[:ROI], preferred_element_type=f32)
            + bl4c[...])
        r2 = jax.nn.relu(jnp.dot(Wl5T[...], r1, preferred_element_type=f32)
                         + bl5c[...])
        allsc = jnp.concatenate([r2, jnp.zeros((NP - ROI, 1), f32)], axis=0)
        alls_row = jnp.sum(jnp.where(eye, allsc, nil), axis=0, keepdims=True)
        Dfu = jnp.where(rr == cc + 4, alls_row, nil)

        A3 = jnp.concatenate(
            [jnp.concatenate([A1, zblk], axis=1),
             jnp.concatenate([Dfu, A2], axis=1)], axis=0)

        zall = jnp.concatenate([z1, z2], axis=0)
        h3 = jnp.dot(zall, W2[...], preferred_element_type=f32)
        xc = jax.nn.relu(_prop(A3, h3) + b2r[...])

        z_o[0, i] = jnp.concatenate(
            [jnp.dot(Se, z1, preferred_element_type=f32),
             jnp.dot(So, z1, preferred_element_type=f32)], axis=1)
        z_o[1, i] = jnp.concatenate(
            [jnp.dot(S2e, z2, preferred_element_type=f32),
             jnp.dot(S2o, z2, preferred_element_type=f32)], axis=1)
        xc_o[i] = jnp.concatenate(
            [jnp.dot(C3e, xc, preferred_element_type=f32),
             jnp.dot(C3o, xc, preferred_element_type=f32)], axis=1)
        alls_o[i] = alls_row


def _head_body(z_ref, w_ref, b_ref, w3_ref, b3_ref, o_ref):
    f32 = jnp.float32
    H = jax.nn.relu(jnp.dot(z_ref[...], w_ref[...], preferred_element_type=f32)
                    + b_ref[...])
    L = jnp.dot(H, w3_ref[...], preferred_element_type=f32) + b3_ref[...]
    m = jnp.max(L, axis=1, keepdims=True)
    e = jnp.exp(L - m)
    o_ref[...] = e / jnp.sum(e, axis=1, keepdims=True)


def _h3_body(x_ref, w_ref, b6_ref, w7_ref, b7_ref, o_ref):
    f32 = jnp.float32
    k = pl.program_id(0)

    @pl.when(k == 0)
    def _():
        o_ref[...] = jnp.zeros_like(o_ref)

    o_ref[...] += jnp.dot(x_ref[...], w_ref[...], preferred_element_type=f32)

    @pl.when(k == pl.num_programs(0) - 1)
    def _():
        H = jax.nn.relu(o_ref[...] + b6_ref[...])
        L = jnp.dot(H, w7_ref[...], preferred_element_type=f32) + b7_ref[...]
        m = jnp.max(L, axis=1, keepdims=True)
        e = jnp.exp(L - m)
        o_ref[...] = e / jnp.sum(e, axis=1, keepdims=True)


def _softmax_head(Z, W, b, W3, b3):
    K = W.shape[0]
    W3p = jnp.pad(W3, ((0, 0), (0, HC2 - W3.shape[1])))
    b3p = jnp.pad(b3, (0, HC2 - b3.shape[0]), constant_values=NEG
                  ).reshape(1, HC2)
    return pl.pallas_call(
        _head_body,
        grid=(1,),
        in_specs=[
            pl.BlockSpec((BG, K), lambda k: (0, 0)),
            pl.BlockSpec((K, HC2), lambda k: (0, 0)),
            pl.BlockSpec((1, HC2), lambda k: (0, 0)),
            pl.BlockSpec((HC2, HC2), lambda k: (0, 0)),
            pl.BlockSpec((1, HC2), lambda k: (0, 0)),
        ],
        out_specs=pl.BlockSpec((BG, HC2), lambda k: (0, 0)),
        out_shape=jax.ShapeDtypeStruct((BG, HC2), jnp.float32),
    )(Z, W, b.reshape(1, HC2), W3p, b3p)


def kernel(x, edge_index, edge_weight, roi_num, batch, device, W1, b1, W11,
           b11, W2, b2, w4, b4, Wl1, bl1, Wl3, bl3, Wl11, bl11, Wl33, bl33,
           Wl4, bl4, Wl5, bl5, Wl6, bl6, Wl7, bl7):
    del roi_num, batch, device
    f32 = jnp.float32

    ei = edge_index.astype(jnp.int32)
    zero_row = jnp.zeros((AW,), f32)
    a_flat = _sc_build_fn()(ei, edge_weight, zero_row)
    ab = a_flat.reshape(2, BG, NP, NP)

    cst = lambda *shape: pl.BlockSpec(shape, lambda g: (0,) * len(shape))
    zo, xco, allso = pl.pallas_call(
        _tc1_body,
        grid=(BG // GB,),
        in_specs=[
            pl.BlockSpec((GB * 2 * ROI, ROI - 1), lambda g: (g, 0)),
            pl.BlockSpec((2, GB, NP, NP), lambda g: (0, g, 0, 0)),
            cst(ROI - 1, HID), cst(ROI - 1, HID), cst(HID, HID),
            cst(2 * HID, 1), cst(1, 1), cst(1, HID), cst(1, HID),
            cst(1, HID), cst(HID, ROI), cst(HID, 1), cst(ROI, HID),
            cst(ROI, 1),
        ],
        out_specs=[
            pl.BlockSpec((2, GB, HID, NP), lambda g: (0, g, 0, 0)),
            pl.BlockSpec((GB, NP, NP), lambda g: (g, 0, 0)),
            pl.BlockSpec((GB, 1, NP), lambda g: (g, 0, 0)),
        ],
        out_shape=[
            jax.ShapeDtypeStruct((2, BG, HID, NP), f32),
            jax.ShapeDtypeStruct((BG, NP, NP), f32),
            jax.ShapeDtypeStruct((BG, 1, NP), f32),
        ],
    )(x, ab, W1, W11, W2, w4.reshape(2 * HID, 1), b4.reshape(1, 1),
      b1.reshape(1, HID), b11.reshape(1, HID), b2.reshape(1, HID),
      Wl4.T, bl4.reshape(HID, 1), Wl5.T, bl5.reshape(ROI, 1))

    x1 = _softmax_head(zo[0].reshape(BG, NP * HID), Wl1, bl1, Wl3, bl3)
    x2 = _softmax_head(zo[1].reshape(BG, NP * HID), Wl11, bl11, Wl33, bl33)

    XF = 2 * ROI * HID
    KCH = 4
    KB = XF // KCH
    Wl7p = jnp.pad(Wl7, ((0, 0), (0, HC2 - 2)))
    bl7p = jnp.pad(bl7, (0, HC2 - 2), constant_values=NEG).reshape(1, HC2)
    xfp = pl.pallas_call(
        _h3_body,
        grid=(KCH,),
        in_specs=[
            pl.BlockSpec((BG, KB), lambda k: (0, k)),
            pl.BlockSpec((KB, HC2), lambda k: (k, 0)),
            pl.BlockSpec((1, HC2), lambda k: (0, 0)),
            pl.BlockSpec((HC2, HC2), lambda k: (0, 0)),
            pl.BlockSpec((1, HC2), lambda k: (0, 0)),
        ],
        out_specs=pl.BlockSpec((BG, HC2), lambda k: (0, 0)),
        out_shape=jax.ShapeDtypeStruct((BG, HC2), f32),
    )(xco.reshape(BG, NP * NP), Wl6, bl6.reshape(1, HC2), Wl7p, bl7p)

    xf = xfp[:, :2]
    x1 = x1[:, :2]
    x2 = x2[:, :2]
    alls = allso.reshape(BG, NP)[:, :ROI]
    return (xf, x1, x2, alls)


# speedup vs baseline: 292.2716x; 1.1408x over previous
import functools

import jax
import jax.numpy as jnp
from jax import lax
from jax.experimental import pallas as pl
from jax.experimental.pallas import tpu as pltpu
from jax.experimental.pallas import tpu_sc as plsc

ROI = 116
LENN = 6670
BG = 128
SEG2 = 2 * LENN + ROI
NE = BG * SEG2
HID = 64
HC2 = 128
NP = 128
N2P = 2 * NP
ECH = 6688
AW = NP * NP
NTASK = 2 * BG
NC, NS = 2, 16
NW = NC * NS
TPW = NTASK // NW
GB = 4
NEG = -1e30


@functools.cache
def _sc_build_fn():
    mesh = plsc.VectorSubcoreMesh(core_axis_name="c", subcore_axis_name="s",
                                  num_cores=NC, num_subcores=NS)

    @functools.partial(
        pl.kernel,
        out_type=jax.ShapeDtypeStruct((NTASK, AW), jnp.float32),
        mesh=mesh,
        scratch_types=[
            pltpu.VMEM((2, ECH), jnp.int32), pltpu.VMEM((2, ECH), jnp.int32),
            pltpu.VMEM((ECH,), jnp.float32), pltpu.VMEM((ECH,), jnp.float32),
            pltpu.VMEM((AW,), jnp.float32), pltpu.VMEM((AW,), jnp.float32),
            pltpu.SemaphoreType.DMA, pltpu.SemaphoreType.DMA,
            pltpu.SemaphoreType.DMA, pltpu.SemaphoreType.DMA,
        ],
        compiler_params=pltpu.CompilerParams(needs_layout_passes=False,
                                             use_tc_tiling_on_sc=False),
    )
    def _sc_build(ei_hbm, ew_hbm, out_hbm,
                  sd0, sd1, w0, w1, a0, a1, is0, is1, os0, os1):
        wid = lax.axis_index("s") * NC + lax.axis_index("c")
        SD, WV, AC = (sd0, sd1), (w0, w1), (a0, a1)
        IS, OS = (is0, is1), (os0, os1)

        def pgab(r):
            t = wid * TPW + r
            p = t // BG
            g = t - p * BG
            abase = pl.multiple_of(g * SEG2 + p * LENN - 6 * p, 8)
            return t, p, g, abase

        def issue_in(r, b):
            _, _, _, abase = pgab(r)
            return (
                pltpu.async_copy(ei_hbm.at[:, pl.ds(abase, ECH)], SD[b], IS[b]),
                pltpu.async_copy(ew_hbm.at[pl.ds(abase, ECH)], WV[b], IS[b]),
            )

        in_d = {0: issue_in(0, 0)}
        out_d = [None, None]
        for r in range(TPW):
            b = r % 2
            t, p, g, _ = pgab(r)
            for d in in_d.pop(r):
                d.wait()
            if r + 1 < TPW:
                in_d[r + 1] = issue_in(r + 1, 1 - b)
            if out_d[b] is not None:
                out_d[b].wait()
            acc = AC[b]
            sd = SD[b]
            wv = WV[b]

            @plsc.parallel_loop(0, AW // 16, 1, unroll=8)
            def zero(i):
                acc[pl.ds(i * 16, 16)] = jnp.zeros((16,), jnp.float32)

            cc = 129 * (g * (2 * ROI) + ROI * p) - 516 * p
            lo = 6 * p
            hi = lo + LENN

            @plsc.parallel_loop(0, ECH // 16, 1, unroll=8)
            def scat(i):
                sl = pl.ds(i * 16, 16)
                pos = i * 16 + lax.iota(jnp.int32, 16)
                valid = (pos >= lo) & (pos < hi)
                idx = sd[1, sl] * NP + sd[0, sl] - cc
                idx = jnp.where(valid, idx, 0)
                w = jnp.where(valid, wv[sl], 0.0)
                plsc.addupdate_scatter(acc, [idx], w)

            out_d[b] = pltpu.async_copy(acc, out_hbm.at[t], OS[b])
        for b in range(2):
            if out_d[b] is not None:
                out_d[b].wait()

    return _sc_build


def _prop(A, h):
    deg = jnp.sum(A, axis=1, keepdims=True) + 1.0
    dinv = lax.rsqrt(deg)
    hs = dinv * h
    t = jnp.dot(A, hs, preferred_element_type=jnp.float32)
    return dinv * (t + hs)


def _tc1_body(x2d, ab, W1, W11, W2, w4c, b4s, b1r, b11r, b2r, Wl4T, bl4c,
              Wl5T, bl5c, z_o, xc_o, alls_o):
    f32 = jnp.float32
    one = jnp.float32(1.0)
    nil = jnp.float32(0.0)
    rr = lax.broadcasted_iota(jnp.int32, (NP, NP), 0)
    cc = lax.broadcasted_iota(jnp.int32, (NP, NP), 1)
    eye = rr == cc
    zblk = jnp.zeros((NP, NP), f32)
    hpad = jnp.zeros((8, HID), f32)
    rih = lax.broadcasted_iota(jnp.int32, (NP, HID), 0)
    rmask1 = rih < ROI
    rmask2 = (rih >= 4) & (rih < 4 + ROI)
    r6 = lax.broadcasted_iota(jnp.int32, (HID, NP), 0)
    c6 = lax.broadcasted_iota(jnp.int32, (HID, NP), 1)
    Se = jnp.where(c6 == 2 * r6, one, nil)
    So = jnp.where(c6 == 2 * r6 + 1, one, nil)
    S2e = jnp.where(c6 == 2 * r6 + 4, one, nil)
    S2o = jnp.where(c6 == 2 * r6 + 5, one, nil)
    r3 = lax.broadcasted_iota(jnp.int32, (NP, N2P), 0)
    c3 = lax.broadcasted_iota(jnp.int32, (NP, N2P), 1)
    sel3 = 2 * r3 + jnp.where(r3 >= ROI // 2, 16, 0)
    C3e = jnp.where(c3 == sel3, one, nil)
    C3o = jnp.where(c3 == sel3 + 1, one, nil)
    Sh4u = jnp.where(cc == rr + 4, one, nil)

    for i in range(GB):
        A1 = ab[0, i]
        A2 = ab[1, i]
        xf_blk = x2d[pl.ds(232 * i, 120), :]
        xs_blk = x2d[pl.ds(232 * i + 112, 120), :]
        h1 = jnp.concatenate(
            [jnp.dot(xf_blk, W1[...], preferred_element_type=f32), hpad],
            axis=0)
        h2 = jnp.concatenate(
            [jnp.dot(xs_blk, W11[...], preferred_element_type=f32), hpad],
            axis=0)
        z1 = jnp.where(rmask1, jax.nn.relu(_prop(A1, h1) + b1r[...]), nil)
        z2 = jnp.where(rmask2, jax.nn.relu(_prop(A2, h2) + b11r[...]), nil)
        z2c = jnp.dot(Sh4u, z2, preferred_element_type=f32)

        zcat = jnp.concatenate([z1, z2c], axis=1)
        alls0 = jnp.dot(zcat, w4c[...], preferred_element_type=f32) + b4s[...]
        r1 = jax.nn.relu(
            jnp.dot(Wl4T[...], alls0[:ROI], preferred_element_type=f32)
            + bl4c[...])
        r2 = jax.nn.relu(jnp.dot(Wl5T[...], r1, preferred_element_type=f32)
                         + bl5c[...])
        allsc = jnp.concatenate([r2, jnp.zeros((NP - ROI, 1), f32)], axis=0)
        alls_row = jnp.sum(jnp.where(eye, allsc, nil), axis=0, keepdims=True)
        Dfu = jnp.where(rr == cc + 4, alls_row, nil)

        A3 = jnp.concatenate(
            [jnp.concatenate([A1, zblk], axis=1),
             jnp.concatenate([Dfu, A2], axis=1)], axis=0)

        zall = jnp.concatenate([z1, z2], axis=0)
        h3 = jnp.dot(zall, W2[...], preferred_element_type=f32)
        xc = jax.nn.relu(_prop(A3, h3) + b2r[...])

        z_o[0, i] = jnp.concatenate(
            [jnp.dot(Se, z1, preferred_element_type=f32),
             jnp.dot(So, z1, preferred_element_type=f32)], axis=1)
        z_o[1, i] = jnp.concatenate(
            [jnp.dot(S2e, z2, preferred_element_type=f32),
             jnp.dot(S2o, z2, preferred_element_type=f32)], axis=1)
        xc_o[i] = jnp.concatenate(
            [jnp.dot(C3e, xc, preferred_element_type=f32),
             jnp.dot(C3o, xc, preferred_element_type=f32)], axis=1)
        alls_o[i] = alls_row


def _head_body(z_ref, w_ref, b_ref, w3_ref, b3_ref, o_ref):
    f32 = jnp.float32
    H = jax.nn.relu(jnp.dot(z_ref[...], w_ref[...], preferred_element_type=f32)
                    + b_ref[...])
    L = jnp.dot(H, w3_ref[...], preferred_element_type=f32) + b3_ref[...]
    m = jnp.max(L, axis=1, keepdims=True)
    e = jnp.exp(L - m)
    o_ref[...] = e / jnp.sum(e, axis=1, keepdims=True)


def _h3_body(x_ref, w_ref, b6_ref, w7_ref, b7_ref, o_ref):
    f32 = jnp.float32
    k = pl.program_id(0)

    @pl.when(k == 0)
    def _():
        o_ref[...] = jnp.zeros_like(o_ref)

    o_ref[...] += jnp.dot(x_ref[...], w_ref[...], preferred_element_type=f32)

    @pl.when(k == pl.num_programs(0) - 1)
    def _():
        H = jax.nn.relu(o_ref[...] + b6_ref[...])
        L = jnp.dot(H, w7_ref[...], preferred_element_type=f32) + b7_ref[...]
        m = jnp.max(L, axis=1, keepdims=True)
        e = jnp.exp(L - m)
        o_ref[...] = e / jnp.sum(e, axis=1, keepdims=True)


def _softmax_head(Z, W, b, W3, b3):
    K = W.shape[0]
    W3p = jnp.pad(W3, ((0, 0), (0, HC2 - W3.shape[1])))
    b3p = jnp.pad(b3, (0, HC2 - b3.shape[0]), constant_values=NEG
                  ).reshape(1, HC2)
    return pl.pallas_call(
        _head_body,
        grid=(1,),
        in_specs=[
            pl.BlockSpec((BG, K), lambda k: (0, 0)),
            pl.BlockSpec((K, HC2), lambda k: (0, 0)),
            pl.BlockSpec((1, HC2), lambda k: (0, 0)),
            pl.BlockSpec((HC2, HC2), lambda k: (0, 0)),
            pl.BlockSpec((1, HC2), lambda k: (0, 0)),
        ],
        out_specs=pl.BlockSpec((BG, HC2), lambda k: (0, 0)),
        out_shape=jax.ShapeDtypeStruct((BG, HC2), jnp.float32),
    )(Z, W, b.reshape(1, HC2), W3p, b3p)


def kernel(x, edge_index, edge_weight, roi_num, batch, device, W1, b1, W11,
           b11, W2, b2, w4, b4, Wl1, bl1, Wl3, bl3, Wl11, bl11, Wl33, bl33,
           Wl4, bl4, Wl5, bl5, Wl6, bl6, Wl7, bl7):
    del roi_num, batch, device
    f32 = jnp.float32

    ei = edge_index.astype(jnp.int32)
    a_flat = _sc_build_fn()(ei, edge_weight)
    ab = a_flat.reshape(2, BG, NP, NP)

    cst = lambda *shape: pl.BlockSpec(shape, lambda g: (0,) * len(shape))
    zo, xco, allso = pl.pallas_call(
        _tc1_body,
        grid=(BG // GB,),
        in_specs=[
            pl.BlockSpec((GB * 2 * ROI, ROI - 1), lambda g: (g, 0)),
            pl.BlockSpec((2, GB, NP, NP), lambda g: (0, g, 0, 0)),
            cst(ROI - 1, HID), cst(ROI - 1, HID), cst(HID, HID),
            cst(2 * HID, 1), cst(1, 1), cst(1, HID), cst(1, HID),
            cst(1, HID), cst(HID, ROI), cst(HID, 1), cst(ROI, HID),
            cst(ROI, 1),
        ],
        out_specs=[
            pl.BlockSpec((2, GB, HID, NP), lambda g: (0, g, 0, 0)),
            pl.BlockSpec((GB, NP, NP), lambda g: (g, 0, 0)),
            pl.BlockSpec((GB, 1, NP), lambda g: (g, 0, 0)),
        ],
        out_shape=[
            jax.ShapeDtypeStruct((2, BG, HID, NP), f32),
            jax.ShapeDtypeStruct((BG, NP, NP), f32),
            jax.ShapeDtypeStruct((BG, 1, NP), f32),
        ],
    )(x, ab, W1, W11, W2, w4.reshape(2 * HID, 1), b4.reshape(1, 1),
      b1.reshape(1, HID), b11.reshape(1, HID), b2.reshape(1, HID),
      Wl4.T, bl4.reshape(HID, 1), Wl5.T, bl5.reshape(ROI, 1))

    x1 = _softmax_head(zo[0].reshape(BG, NP * HID), Wl1, bl1, Wl3, bl3)
    x2 = _softmax_head(zo[1].reshape(BG, NP * HID), Wl11, bl11, Wl33, bl33)

    XF = 2 * ROI * HID
    KCH = 4
    KB = XF // KCH
    Wl7p = jnp.pad(Wl7, ((0, 0), (0, HC2 - 2)))
    bl7p = jnp.pad(bl7, (0, HC2 - 2), constant_values=NEG).reshape(1, HC2)
    xfp = pl.pallas_call(
        _h3_body,
        grid=(KCH,),
        in_specs=[
            pl.BlockSpec((BG, KB), lambda k: (0, k)),
            pl.BlockSpec((KB, HC2), lambda k: (k, 0)),
            pl.BlockSpec((1, HC2), lambda k: (0, 0)),
            pl.BlockSpec((HC2, HC2), lambda k: (0, 0)),
            pl.BlockSpec((1, HC2), lambda k: (0, 0)),
        ],
        out_specs=pl.BlockSpec((BG, HC2), lambda k: (0, 0)),
        out_shape=jax.ShapeDtypeStruct((BG, HC2), f32),
    )(xco.reshape(BG, NP * NP), Wl6, bl6.reshape(1, HC2), Wl7p, bl7p)

    xf = xfp[:, :2]
    x1 = x1[:, :2]
    x2 = x2[:, :2]
    alls = allso.reshape(BG, NP)[:, :ROI]
    return (xf, x1, x2, alls)


# speedup vs baseline: 555.0334x; 1.8990x over previous
import functools

import jax
import jax.numpy as jnp
from jax import lax
from jax.experimental import pallas as pl
from jax.experimental.pallas import tpu as pltpu
from jax.experimental.pallas import tpu_sc as plsc

ROI = 116
LENN = 6670
BG = 128
SEG2 = 2 * LENN + ROI
NE = BG * SEG2
HID = 64
HC2 = 128
NP = 128
N2P = 2 * NP
ECH = 6688
AW = NP * NP
NTASK = 2 * BG
NC, NS = 2, 16
NW = NC * NS
TPW = NTASK // NW
GB = 8
NEG = -1e30


@functools.cache
def _sc_build_fn():
    mesh = plsc.VectorSubcoreMesh(core_axis_name="c", subcore_axis_name="s",
                                  num_cores=NC, num_subcores=NS)

    @functools.partial(
        pl.kernel,
        out_type=jax.ShapeDtypeStruct((NTASK, AW), jnp.float32),
        mesh=mesh,
        scratch_types=[
            pltpu.VMEM((2, ECH), jnp.int32), pltpu.VMEM((2, ECH), jnp.int32),
            pltpu.VMEM((ECH,), jnp.float32), pltpu.VMEM((ECH,), jnp.float32),
            pltpu.VMEM((AW,), jnp.float32), pltpu.VMEM((AW,), jnp.float32),
            pltpu.SemaphoreType.DMA, pltpu.SemaphoreType.DMA,
            pltpu.SemaphoreType.DMA, pltpu.SemaphoreType.DMA,
        ],
        compiler_params=pltpu.CompilerParams(needs_layout_passes=False,
                                             use_tc_tiling_on_sc=False),
    )
    def _sc_build(ei_hbm, ew_hbm, out_hbm,
                  sd0, sd1, w0, w1, a0, a1, is0, is1, os0, os1):
        wid = lax.axis_index("s") * NC + lax.axis_index("c")
        SD, WV, AC = (sd0, sd1), (w0, w1), (a0, a1)
        IS, OS = (is0, is1), (os0, os1)

        def pgab(r):
            t = wid * TPW + r
            p = t // BG
            g = t - p * BG
            abase = pl.multiple_of(g * SEG2 + p * LENN - 6 * p, 8)
            return t, p, g, abase

        def issue_in(r, b):
            _, _, _, abase = pgab(r)
            return (
                pltpu.async_copy(ei_hbm.at[:, pl.ds(abase, ECH)], SD[b], IS[b]),
                pltpu.async_copy(ew_hbm.at[pl.ds(abase, ECH)], WV[b], IS[b]),
            )

        in_d = {0: issue_in(0, 0)}
        out_d = [None, None]
        for r in range(TPW):
            b = r % 2
            t, p, g, _ = pgab(r)
            for d in in_d.pop(r):
                d.wait()
            if r + 1 < TPW:
                in_d[r + 1] = issue_in(r + 1, 1 - b)
            if out_d[b] is not None:
                out_d[b].wait()
            acc = AC[b]
            sd = SD[b]
            wv = WV[b]

            @plsc.parallel_loop(0, AW // 16, 1, unroll=8)
            def zero(i):
                acc[pl.ds(i * 16, 16)] = jnp.zeros((16,), jnp.float32)

            cc = 129 * (g * (2 * ROI) + ROI * p) - 516 * p
            lo = 6 * p
            hi = lo + LENN

            @plsc.parallel_loop(0, ECH // 16, 1, unroll=8)
            def scat(i):
                sl = pl.ds(i * 16, 16)
                pos = i * 16 + lax.iota(jnp.int32, 16)
                valid = (pos >= lo) & (pos < hi)
                idx = sd[1, sl] * NP + sd[0, sl] - cc
                idx = jnp.where(valid, idx, 0)
                w = jnp.where(valid, wv[sl], 0.0)
                plsc.addupdate_scatter(acc, [idx], w)

            out_d[b] = pltpu.async_copy(acc, out_hbm.at[t], OS[b])
        for b in range(2):
            if out_d[b] is not None:
                out_d[b].wait()

    return _sc_build


def _prop(A, h):
    deg = jnp.sum(A, axis=1, keepdims=True) + 1.0
    dinv = lax.rsqrt(deg)
    hs = dinv * h
    t = jnp.dot(A, hs, preferred_element_type=jnp.float32)
    return dinv * (t + hs)


def _tc1_body(x2d, ab, W1, W11, W2, w4c, b4s, b1r, b11r, b2r, Wl4T, bl4c,
              Wl5T, bl5c, z_o, xc_o, alls_o):
    f32 = jnp.float32
    one = jnp.float32(1.0)
    nil = jnp.float32(0.0)
    rr = lax.broadcasted_iota(jnp.int32, (NP, NP), 0)
    cc = lax.broadcasted_iota(jnp.int32, (NP, NP), 1)
    eye = rr == cc
    zblk = jnp.zeros((NP, NP), f32)
    hpad = jnp.zeros((8, HID), f32)
    rih = lax.broadcasted_iota(jnp.int32, (NP, HID), 0)
    rmask1 = rih < ROI
    rmask2 = (rih >= 4) & (rih < 4 + ROI)
    r6 = lax.broadcasted_iota(jnp.int32, (HID, NP), 0)
    c6 = lax.broadcasted_iota(jnp.int32, (HID, NP), 1)
    Se = jnp.where(c6 == 2 * r6, one, nil)
    So = jnp.where(c6 == 2 * r6 + 1, one, nil)
    S2e = jnp.where(c6 == 2 * r6 + 4, one, nil)
    S2o = jnp.where(c6 == 2 * r6 + 5, one, nil)
    r3 = lax.broadcasted_iota(jnp.int32, (NP, N2P), 0)
    c3 = lax.broadcasted_iota(jnp.int32, (NP, N2P), 1)
    sel3 = 2 * r3 + jnp.where(r3 >= ROI // 2, 16, 0)
    C3e = jnp.where(c3 == sel3, one, nil)
    C3o = jnp.where(c3 == sel3 + 1, one, nil)
    Sh4u = jnp.where(cc == rr + 4, one, nil)

    R = range(GB)
    mm = lambda a, b: jnp.dot(a, b, preferred_element_type=f32)
    A1 = [ab[0, i] for i in R]
    A2 = [ab[1, i] for i in R]
    h1 = [jnp.concatenate(
        [mm(x2d[pl.ds(232 * i, 120), :], W1[...]), hpad], axis=0) for i in R]
    h2 = [jnp.concatenate(
        [mm(x2d[pl.ds(232 * i + 112, 120), :], W11[...]), hpad], axis=0)
        for i in R]
    d1 = [lax.rsqrt(jnp.sum(A1[i], axis=1, keepdims=True) + 1.0) for i in R]
    d2 = [lax.rsqrt(jnp.sum(A2[i], axis=1, keepdims=True) + 1.0) for i in R]
    hs1 = [d1[i] * h1[i] for i in R]
    hs2 = [d2[i] * h2[i] for i in R]
    t1 = [mm(A1[i], hs1[i]) for i in R]
    t2 = [mm(A2[i], hs2[i]) for i in R]
    z1 = [jnp.where(rmask1,
                    jax.nn.relu(d1[i] * (t1[i] + hs1[i]) + b1r[...]), nil)
          for i in R]
    z2 = [jnp.where(rmask2,
                    jax.nn.relu(d2[i] * (t2[i] + hs2[i]) + b11r[...]), nil)
          for i in R]
    z2c = [mm(Sh4u, z2[i]) for i in R]
    zcat = [jnp.concatenate([z1[i], z2c[i]], axis=1) for i in R]
    alls0 = [mm(zcat[i], w4c[...]) + b4s[...] for i in R]
    r1 = [jax.nn.relu(mm(Wl4T[...], alls0[i][:ROI]) + bl4c[...]) for i in R]
    r2 = [jax.nn.relu(mm(Wl5T[...], r1[i]) + bl5c[...]) for i in R]
    allsc = [jnp.concatenate([r2[i], jnp.zeros((NP - ROI, 1), f32)], axis=0)
             for i in R]
    arow = [jnp.sum(jnp.where(eye, allsc[i], nil), axis=0, keepdims=True)
            for i in R]
    Dfu = [jnp.where(rr == cc + 4, arow[i], nil) for i in R]
    A3 = [jnp.concatenate(
        [jnp.concatenate([A1[i], zblk], axis=1),
         jnp.concatenate([Dfu[i], A2[i]], axis=1)], axis=0) for i in R]
    zall = [jnp.concatenate([z1[i], z2[i]], axis=0) for i in R]
    h3 = [mm(zall[i], W2[...]) for i in R]
    d3 = [lax.rsqrt(jnp.sum(A3[i], axis=1, keepdims=True) + 1.0) for i in R]
    hs3 = [d3[i] * h3[i] for i in R]
    t3 = [mm(A3[i], hs3[i]) for i in R]
    xc = [jax.nn.relu(d3[i] * (t3[i] + hs3[i]) + b2r[...]) for i in R]

    for i in R:
        z_o[0, i] = jnp.concatenate(
            [mm(Se, z1[i]), mm(So, z1[i])], axis=1)
        z_o[1, i] = jnp.concatenate(
            [mm(S2e, z2[i]), mm(S2o, z2[i])], axis=1)
        xc_o[i] = jnp.concatenate(
            [mm(C3e, xc[i]), mm(C3o, xc[i])], axis=1)
        alls_o[i] = arow[i]


def _head_body(z_ref, w_ref, b_ref, w3_ref, b3_ref, o_ref):
    f32 = jnp.float32
    H = jax.nn.relu(jnp.dot(z_ref[...], w_ref[...], preferred_element_type=f32)
                    + b_ref[...])
    L = jnp.dot(H, w3_ref[...], preferred_element_type=f32) + b3_ref[...]
    m = jnp.max(L, axis=1, keepdims=True)
    e = jnp.exp(L - m)
    o_ref[...] = e / jnp.sum(e, axis=1, keepdims=True)


def _h3_body(x_ref, w_ref, b6_ref, w7_ref, b7_ref, o_ref):
    f32 = jnp.float32
    k = pl.program_id(0)

    @pl.when(k == 0)
    def _():
        o_ref[...] = jnp.zeros_like(o_ref)

    o_ref[...] += jnp.dot(x_ref[...], w_ref[...], preferred_element_type=f32)

    @pl.when(k == pl.num_programs(0) - 1)
    def _():
        H = jax.nn.relu(o_ref[...] + b6_ref[...])
        L = jnp.dot(H, w7_ref[...], preferred_element_type=f32) + b7_ref[...]
        m = jnp.max(L, axis=1, keepdims=True)
        e = jnp.exp(L - m)
        o_ref[...] = e / jnp.sum(e, axis=1, keepdims=True)


def _softmax_head(Z, W, b, W3, b3):
    K = W.shape[0]
    W3p = jnp.pad(W3, ((0, 0), (0, HC2 - W3.shape[1])))
    b3p = jnp.pad(b3, (0, HC2 - b3.shape[0]), constant_values=NEG
                  ).reshape(1, HC2)
    return pl.pallas_call(
        _head_body,
        grid=(1,),
        in_specs=[
            pl.BlockSpec((BG, K), lambda k: (0, 0)),
            pl.BlockSpec((K, HC2), lambda k: (0, 0)),
            pl.BlockSpec((1, HC2), lambda k: (0, 0)),
            pl.BlockSpec((HC2, HC2), lambda k: (0, 0)),
            pl.BlockSpec((1, HC2), lambda k: (0, 0)),
        ],
        out_specs=pl.BlockSpec((BG, HC2), lambda k: (0, 0)),
        out_shape=jax.ShapeDtypeStruct((BG, HC2), jnp.float32),
    )(Z, W, b.reshape(1, HC2), W3p, b3p)


def kernel(x, edge_index, edge_weight, roi_num, batch, device, W1, b1, W11,
           b11, W2, b2, w4, b4, Wl1, bl1, Wl3, bl3, Wl11, bl11, Wl33, bl33,
           Wl4, bl4, Wl5, bl5, Wl6, bl6, Wl7, bl7):
    del roi_num, batch, device
    f32 = jnp.float32

    ei = edge_index.astype(jnp.int32)
    a_flat = _sc_build_fn()(ei, edge_weight)
    ab = a_flat.reshape(2, BG, NP, NP)

    cst = lambda *shape: pl.BlockSpec(shape, lambda g: (0,) * len(shape))
    zo, xco, allso = pl.pallas_call(
        _tc1_body,
        grid=(BG // GB,),
        in_specs=[
            pl.BlockSpec((GB * 2 * ROI, ROI - 1), lambda g: (g, 0)),
            pl.BlockSpec((2, GB, NP, NP), lambda g: (0, g, 0, 0)),
            cst(ROI - 1, HID), cst(ROI - 1, HID), cst(HID, HID),
            cst(2 * HID, 1), cst(1, 1), cst(1, HID), cst(1, HID),
            cst(1, HID), cst(HID, ROI), cst(HID, 1), cst(ROI, HID),
            cst(ROI, 1),
        ],
        out_specs=[
            pl.BlockSpec((2, GB, HID, NP), lambda g: (0, g, 0, 0)),
            pl.BlockSpec((GB, NP, NP), lambda g: (g, 0, 0)),
            pl.BlockSpec((GB, 1, NP), lambda g: (g, 0, 0)),
        ],
        out_shape=[
            jax.ShapeDtypeStruct((2, BG, HID, NP), f32),
            jax.ShapeDtypeStruct((BG, NP, NP), f32),
            jax.ShapeDtypeStruct((BG, 1, NP), f32),
        ],
    )(x, ab, W1, W11, W2, w4.reshape(2 * HID, 1), b4.reshape(1, 1),
      b1.reshape(1, HID), b11.reshape(1, HID), b2.reshape(1, HID),
      Wl4.T, bl4.reshape(HID, 1), Wl5.T, bl5.reshape(ROI, 1))

    x1 = _softmax_head(zo[0].reshape(BG, NP * HID), Wl1, bl1, Wl3, bl3)
    x2 = _softmax_head(zo[1].reshape(BG, NP * HID), Wl11, bl11, Wl33, bl33)

    XF = 2 * ROI * HID
    KCH = 4
    KB = XF // KCH
    Wl7p = jnp.pad(Wl7, ((0, 0), (0, HC2 - 2)))
    bl7p = jnp.pad(bl7, (0, HC2 - 2), constant_values=NEG).reshape(1, HC2)
    xfp = pl.pallas_call(
        _h3_body,
        grid=(KCH,),
        in_specs=[
            pl.BlockSpec((BG, KB), lambda k: (0, k)),
            pl.BlockSpec((KB, HC2), lambda k: (k, 0)),
            pl.BlockSpec((1, HC2), lambda k: (0, 0)),
            pl.BlockSpec((HC2, HC2), lambda k: (0, 0)),
            pl.BlockSpec((1, HC2), lambda k: (0, 0)),
        ],
        out_specs=pl.BlockSpec((BG, HC2), lambda k: (0, 0)),
        out_shape=jax.ShapeDtypeStruct((BG, HC2), f32),
    )(xco.reshape(BG, NP * NP), Wl6, bl6.reshape(1, HC2), Wl7p, bl7p)

    xf = xfp[:, :2]
    x1 = x1[:, :2]
    x2 = x2[:, :2]
    alls = allso.reshape(BG, NP)[:, :ROI]
    return (xf, x1, x2, alls)


# speedup vs baseline: 623.3059x; 1.1230x over previous
import functools

import jax
import jax.numpy as jnp
from jax import lax
from jax.experimental import pallas as pl
from jax.experimental.pallas import tpu as pltpu
from jax.experimental.pallas import tpu_sc as plsc

ROI = 116
LENN = 6670
BG = 128
SEG2 = 2 * LENN + ROI
NE = BG * SEG2
HID = 64
HC2 = 128
NP = 128
N2P = 2 * NP
ECH = 6912
AW = NP * NP
NTASK = 2 * BG
NC, NS = 2, 16
NW = NC * NS
TPW = NTASK // NW
GB = 8
NEG = -1e30


@functools.cache
def _sc_build_fn():
    mesh = plsc.VectorSubcoreMesh(core_axis_name="c", subcore_axis_name="s",
                                  num_cores=NC, num_subcores=NS)

    @functools.partial(
        pl.kernel,
        out_type=jax.ShapeDtypeStruct((NTASK, NP, NP), jnp.float32),
        mesh=mesh,
        scratch_types=[
            pltpu.VMEM((2, ECH), jnp.int32), pltpu.VMEM((2, ECH), jnp.int32),
            pltpu.VMEM((ECH,), jnp.float32), pltpu.VMEM((ECH,), jnp.float32),
            pltpu.VMEM((NP, NP), jnp.float32), pltpu.VMEM((NP, NP), jnp.float32),
            pltpu.SemaphoreType.DMA, pltpu.SemaphoreType.DMA,
            pltpu.SemaphoreType.DMA, pltpu.SemaphoreType.DMA,
        ],
        compiler_params=pltpu.CompilerParams(needs_layout_passes=False,
                                             use_tc_tiling_on_sc=True),
    )
    def _sc_build(ei_hbm, ew_hbm, out_hbm,
                  sd0, sd1, w0, w1, a0, a1, is0, is1, os0, os1):
        wid = lax.axis_index("s") * NC + lax.axis_index("c")
        SD, WV, AC = (sd0, sd1), (w0, w1), (a0, a1)
        IS, OS = (is0, is1), (os0, os1)

        def pgab(r):
            t = wid * TPW + r
            p = t // BG
            g = t - p * BG
            base = g * SEG2 + p * LENN
            abase = pl.multiple_of(base - base % 128, 128)
            return t, p, g, base, abase

        def issue_in(r, b):
            _, _, _, _, abase = pgab(r)
            return (
                pltpu.async_copy(ei_hbm.at[:, pl.ds(abase, ECH)], SD[b], IS[b]),
                pltpu.async_copy(ew_hbm.at[pl.ds(abase, ECH)], WV[b], IS[b]),
            )

        in_d = {0: issue_in(0, 0)}
        out_d = [None, None]
        for r in range(TPW):
            b = r % 2
            t, p, g, base, abase = pgab(r)
            for d in in_d.pop(r):
                d.wait()
            if r + 1 < TPW:
                in_d[r + 1] = issue_in(r + 1, 1 - b)
            if out_d[b] is not None:
                out_d[b].wait()
            acc = AC[b]
            sd = SD[b]
            wv = WV[b]

            @plsc.parallel_loop(0, NP * NP // 128, 1, unroll=8)
            def zero(i):
                acc[i // 8, pl.ds((i % 8) * 16, 16)] = jnp.zeros(
                    (16,), jnp.float32)

            csub = g * (2 * ROI) + ROI * p - 4 * p
            lo = base - abase
            hi = lo + LENN

            @plsc.parallel_loop(0, ECH // 16, 1, unroll=8)
            def scat(i):
                sl = pl.ds(i * 16, 16)
                pos = i * 16 + lax.iota(jnp.int32, 16)
                valid = (pos >= lo) & (pos < hi)
                dv = jnp.where(valid, sd[1, sl] - csub, 0)
                sv = jnp.where(valid, sd[0, sl] - csub, 0)
                w = jnp.where(valid, wv[sl], 0.0)
                plsc.addupdate_scatter(acc, [dv, sv], w)

            out_d[b] = pltpu.async_copy(acc, out_hbm.at[t], OS[b])
        for b in range(2):
            if out_d[b] is not None:
                out_d[b].wait()

    return _sc_build


def _prop(A, h):
    deg = jnp.sum(A, axis=1, keepdims=True) + 1.0
    dinv = lax.rsqrt(deg)
    hs = dinv * h
    t = jnp.dot(A, hs, preferred_element_type=jnp.float32)
    return dinv * (t + hs)


def _tc1_body(x2d, ab, W1, W11, W2, w4c, b4s, b1r, b11r, b2r, Wl4T, bl4c,
              Wl5T, bl5c, z_o, xc_o, alls_o):
    f32 = jnp.float32
    one = jnp.float32(1.0)
    nil = jnp.float32(0.0)
    rr = lax.broadcasted_iota(jnp.int32, (NP, NP), 0)
    cc = lax.broadcasted_iota(jnp.int32, (NP, NP), 1)
    eye = rr == cc
    zblk = jnp.zeros((NP, NP), f32)
    hpad = jnp.zeros((8, HID), f32)
    rih = lax.broadcasted_iota(jnp.int32, (NP, HID), 0)
    rmask1 = rih < ROI
    rmask2 = (rih >= 4) & (rih < 4 + ROI)
    r6 = lax.broadcasted_iota(jnp.int32, (HID, NP), 0)
    c6 = lax.broadcasted_iota(jnp.int32, (HID, NP), 1)
    Se = jnp.where(c6 == 2 * r6, one, nil)
    So = jnp.where(c6 == 2 * r6 + 1, one, nil)
    S2e = jnp.where(c6 == 2 * r6 + 4, one, nil)
    S2o = jnp.where(c6 == 2 * r6 + 5, one, nil)
    r3 = lax.broadcasted_iota(jnp.int32, (NP, N2P), 0)
    c3 = lax.broadcasted_iota(jnp.int32, (NP, N2P), 1)
    sel3 = 2 * r3 + jnp.where(r3 >= ROI // 2, 16, 0)
    C3e = jnp.where(c3 == sel3, one, nil)
    C3o = jnp.where(c3 == sel3 + 1, one, nil)
    Sh4u = jnp.where(cc == rr + 4, one, nil)

    R = range(GB)
    mm = lambda a, b: jnp.dot(a, b, preferred_element_type=f32)
    A1 = [ab[0, i] for i in R]
    A2 = [ab[1, i] for i in R]
    h1 = [jnp.concatenate(
        [mm(x2d[pl.ds(232 * i, 120), :], W1[...]), hpad], axis=0) for i in R]
    h2 = [jnp.concatenate(
        [mm(x2d[pl.ds(232 * i + 112, 120), :], W11[...]), hpad], axis=0)
        for i in R]
    d1 = [lax.rsqrt(jnp.sum(A1[i], axis=1, keepdims=True) + 1.0) for i in R]
    d2 = [lax.rsqrt(jnp.sum(A2[i], axis=1, keepdims=True) + 1.0) for i in R]
    hs1 = [d1[i] * h1[i] for i in R]
    hs2 = [d2[i] * h2[i] for i in R]
    t1 = [mm(A1[i], hs1[i]) for i in R]
    t2 = [mm(A2[i], hs2[i]) for i in R]
    z1 = [jnp.where(rmask1,
                    jax.nn.relu(d1[i] * (t1[i] + hs1[i]) + b1r[...]), nil)
          for i in R]
    z2 = [jnp.where(rmask2,
                    jax.nn.relu(d2[i] * (t2[i] + hs2[i]) + b11r[...]), nil)
          for i in R]
    z2c = [mm(Sh4u, z2[i]) for i in R]
    zcat = [jnp.concatenate([z1[i], z2c[i]], axis=1) for i in R]
    alls0 = [mm(zcat[i], w4c[...]) + b4s[...] for i in R]
    r1 = [jax.nn.relu(mm(Wl4T[...], alls0[i][:ROI]) + bl4c[...]) for i in R]
    r2 = [jax.nn.relu(mm(Wl5T[...], r1[i]) + bl5c[...]) for i in R]
    allsc = [jnp.concatenate([r2[i], jnp.zeros((NP - ROI, 1), f32)], axis=0)
             for i in R]
    arow = [jnp.sum(jnp.where(eye, allsc[i], nil), axis=0, keepdims=True)
            for i in R]
    Dfu = [jnp.where(rr == cc + 4, arow[i], nil) for i in R]
    A3 = [jnp.concatenate(
        [jnp.concatenate([A1[i], zblk], axis=1),
         jnp.concatenate([Dfu[i], A2[i]], axis=1)], axis=0) for i in R]
    zall = [jnp.concatenate([z1[i], z2[i]], axis=0) for i in R]
    h3 = [mm(zall[i], W2[...]) for i in R]
    d3 = [lax.rsqrt(jnp.sum(A3[i], axis=1, keepdims=True) + 1.0) for i in R]
    hs3 = [d3[i] * h3[i] for i in R]
    t3 = [mm(A3[i], hs3[i]) for i in R]
    xc = [jax.nn.relu(d3[i] * (t3[i] + hs3[i]) + b2r[...]) for i in R]

    for i in R:
        z_o[0, i] = jnp.concatenate(
            [mm(Se, z1[i]), mm(So, z1[i])], axis=1)
        z_o[1, i] = jnp.concatenate(
            [mm(S2e, z2[i]), mm(S2o, z2[i])], axis=1)
        xc_o[i] = jnp.concatenate(
            [mm(C3e, xc[i]), mm(C3o, xc[i])], axis=1)
        alls_o[i] = arow[i]


def _head_body(z_ref, w_ref, b_ref, w3_ref, b3_ref, o_ref):
    f32 = jnp.float32
    H = jax.nn.relu(jnp.dot(z_ref[...], w_ref[...], preferred_element_type=f32)
                    + b_ref[...])
    L = jnp.dot(H, w3_ref[...], preferred_element_type=f32) + b3_ref[...]
    m = jnp.max(L, axis=1, keepdims=True)
    e = jnp.exp(L - m)
    o_ref[...] = e / jnp.sum(e, axis=1, keepdims=True)


def _h3_body(x_ref, w_ref, b6_ref, w7_ref, b7_ref, o_ref):
    f32 = jnp.float32
    k = pl.program_id(0)

    @pl.when(k == 0)
    def _():
        o_ref[...] = jnp.zeros_like(o_ref)

    o_ref[...] += jnp.dot(x_ref[...], w_ref[...], preferred_element_type=f32)

    @pl.when(k == pl.num_programs(0) - 1)
    def _():
        H = jax.nn.relu(o_ref[...] + b6_ref[...])
        L = jnp.dot(H, w7_ref[...], preferred_element_type=f32) + b7_ref[...]
        m = jnp.max(L, axis=1, keepdims=True)
        e = jnp.exp(L - m)
        o_ref[...] = e / jnp.sum(e, axis=1, keepdims=True)


def _softmax_head(Z, W, b, W3, b3):
    K = W.shape[0]
    W3p = jnp.pad(W3, ((0, 0), (0, HC2 - W3.shape[1])))
    b3p = jnp.pad(b3, (0, HC2 - b3.shape[0]), constant_values=NEG
                  ).reshape(1, HC2)
    return pl.pallas_call(
        _head_body,
        grid=(1,),
        in_specs=[
            pl.BlockSpec((BG, K), lambda k: (0, 0)),
            pl.BlockSpec((K, HC2), lambda k: (0, 0)),
            pl.BlockSpec((1, HC2), lambda k: (0, 0)),
            pl.BlockSpec((HC2, HC2), lambda k: (0, 0)),
            pl.BlockSpec((1, HC2), lambda k: (0, 0)),
        ],
        out_specs=pl.BlockSpec((BG, HC2), lambda k: (0, 0)),
        out_shape=jax.ShapeDtypeStruct((BG, HC2), jnp.float32),
    )(Z, W, b.reshape(1, HC2), W3p, b3p)


def kernel(x, edge_index, edge_weight, roi_num, batch, device, W1, b1, W11,
           b11, W2, b2, w4, b4, Wl1, bl1, Wl3, bl3, Wl11, bl11, Wl33, bl33,
           Wl4, bl4, Wl5, bl5, Wl6, bl6, Wl7, bl7):
    del roi_num, batch, device
    f32 = jnp.float32

    ei = edge_index.astype(jnp.int32)
    a_flat = _sc_build_fn()(ei, edge_weight)
    ab = a_flat.reshape(2, BG, NP, NP)

    cst = lambda *shape: pl.BlockSpec(shape, lambda g: (0,) * len(shape))
    zo, xco, allso = pl.pallas_call(
        _tc1_body,
        grid=(BG // GB,),
        in_specs=[
            pl.BlockSpec((GB * 2 * ROI, ROI - 1), lambda g: (g, 0)),
            pl.BlockSpec((2, GB, NP, NP), lambda g: (0, g, 0, 0)),
            cst(ROI - 1, HID), cst(ROI - 1, HID), cst(HID, HID),
            cst(2 * HID, 1), cst(1, 1), cst(1, HID), cst(1, HID),
            cst(1, HID), cst(HID, ROI), cst(HID, 1), cst(ROI, HID),
            cst(ROI, 1),
        ],
        out_specs=[
            pl.BlockSpec((2, GB, HID, NP), lambda g: (0, g, 0, 0)),
            pl.BlockSpec((GB, NP, NP), lambda g: (g, 0, 0)),
            pl.BlockSpec((GB, 1, NP), lambda g: (g, 0, 0)),
        ],
        out_shape=[
            jax.ShapeDtypeStruct((2, BG, HID, NP), f32),
            jax.ShapeDtypeStruct((BG, NP, NP), f32),
            jax.ShapeDtypeStruct((BG, 1, NP), f32),
        ],
    )(x, ab, W1, W11, W2, w4.reshape(2 * HID, 1), b4.reshape(1, 1),
      b1.reshape(1, HID), b11.reshape(1, HID), b2.reshape(1, HID),
      Wl4.T, bl4.reshape(HID, 1), Wl5.T, bl5.reshape(ROI, 1))

    x1 = _softmax_head(zo[0].reshape(BG, NP * HID), Wl1, bl1, Wl3, bl3)
    x2 = _softmax_head(zo[1].reshape(BG, NP * HID), Wl11, bl11, Wl33, bl33)

    XF = 2 * ROI * HID
    KCH = 4
    KB = XF // KCH
    Wl7p = jnp.pad(Wl7, ((0, 0), (0, HC2 - 2)))
    bl7p = jnp.pad(bl7, (0, HC2 - 2), constant_values=NEG).reshape(1, HC2)
    xfp = pl.pallas_call(
        _h3_body,
        grid=(KCH,),
        in_specs=[
            pl.BlockSpec((BG, KB), lambda k: (0, k)),
            pl.BlockSpec((KB, HC2), lambda k: (k, 0)),
            pl.BlockSpec((1, HC2), lambda k: (0, 0)),
            pl.BlockSpec((HC2, HC2), lambda k: (0, 0)),
            pl.BlockSpec((1, HC2), lambda k: (0, 0)),
        ],
        out_specs=pl.BlockSpec((BG, HC2), lambda k: (0, 0)),
        out_shape=jax.ShapeDtypeStruct((BG, HC2), f32),
    )(xco.reshape(BG, NP * NP), Wl6, bl6.reshape(1, HC2), Wl7p, bl7p)

    xf = xfp[:, :2]
    x1 = x1[:, :2]
    x2 = x2[:, :2]
    alls = allso.reshape(BG, NP)[:, :ROI]
    return (xf, x1, x2, alls)
